# Initial kernel scaffold; baseline (speedup 1.0000x reference)
import jax
import jax.numpy as jnp
from jax.experimental import pallas as pl


def kernel(image, label):
    raise NotImplementedError("write your pallas kernel here")


# speedup vs baseline: 6.8217x; 6.8217x over previous
import functools

import jax
import jax.numpy as jnp
from jax import lax
from jax.experimental import pallas as pl
from jax.experimental.pallas import tpu as pltpu
from jax.experimental.pallas import tpu_sc as plsc

_EMB = 448


def _mean_body(nblk, inv_n, x_ref, o_ref, acc_ref):
    i = pl.program_id(0)

    @pl.when(i == 0)
    def _init():
        acc_ref[...] = jnp.zeros_like(acc_ref)

    x = x_ref[...]
    c, bc = x.shape
    acc_ref[...] += x.reshape(c, bc // 128, 128).sum(axis=1)

    @pl.when(i == nblk - 1)
    def _fin():
        o_ref[...] = acc_ref[...].sum(axis=1, keepdims=True) * inv_n


def _channel_means(x2d):
    c, n = x2d.shape
    bc = 8192
    assert n % bc == 0
    nblk = n // bc
    return pl.pallas_call(
        functools.partial(_mean_body, nblk, 1.0 / n),
        grid=(nblk,),
        in_specs=[pl.BlockSpec((c, bc), lambda i: (0, i))],
        out_specs=pl.BlockSpec((c, 1), lambda i: (0, 0)),
        out_shape=jax.ShapeDtypeStruct((c, 1), jnp.float32),
        scratch_shapes=[pltpu.VMEM((c, 128), jnp.float32)],
    )(x2d)


def _make_hist(n, nbins):
    info = plsc.get_sparse_core_info()
    nc, ns, nl = info.num_cores, info.num_subcores, info.num_lanes
    nw = nc * ns
    per_w = n // nw
    assert n % nw == 0 and per_w % nl == 0 and per_w % 8 == 0
    mesh = plsc.VectorSubcoreMesh(core_axis_name="c", subcore_axis_name="s")

    @functools.partial(
        pl.kernel,
        mesh=mesh,
        compiler_params=pltpu.CompilerParams(needs_layout_passes=False),
        out_type=jax.ShapeDtypeStruct((nw, nbins), jnp.float32),
        scratch_types=[
            pltpu.VMEM((per_w,), jnp.int32),
            pltpu.VMEM((nl * nbins,), jnp.float32),
            pltpu.VMEM((nbins,), jnp.float32),
        ],
    )
    def hist_kernel(lbl_hbm, out_hbm, lbl_v, hist_v, part_v):
        wid = lax.axis_index("s") * nc + lax.axis_index("c")
        base = wid * per_w
        pltpu.sync_copy(lbl_hbm.at[pl.ds(base, per_w)], lbl_v)

        def _zero(t, carry):
            hist_v[pl.ds(t * nl, nl)] = jnp.zeros((nl,), jnp.float32)
            return carry

        lax.fori_loop(0, (nl * nbins) // nl, _zero, 0)

        lane_base = lax.iota(jnp.int32, nl) * nbins
        ones = jnp.ones((nl,), jnp.float32)

        def _scat(j, carry):
            idx = lbl_v[pl.ds(j * nl, nl)]
            plsc.addupdate_scatter(hist_v, [lane_base + idx], ones)
            return carry

        lax.fori_loop(0, per_w // nl, _scat, 0)

        for cchunk in range(nbins // nl):
            acc = jnp.zeros((nl,), jnp.float32)
            for l in range(nl):
                acc = acc + hist_v[pl.ds(l * nbins + cchunk * nl, nl)]
            part_v[pl.ds(cchunk * nl, nl)] = acc

        pltpu.sync_copy(part_v, out_hbm.at[wid])

    return hist_kernel


def kernel(image, label):
    c, h, w = image.shape
    n = h * w
    nbins = _EMB - c
    mean_c = _channel_means(image.reshape(c, n))
    parts = _make_hist(n, nbins)(label.reshape(n))
    hist = parts.sum(axis=0) * (1.0 / n)
    return jnp.concatenate([mean_c[:, 0], hist])


# speedup vs baseline: 16.2588x; 2.3834x over previous
import functools

import jax
import jax.numpy as jnp
from jax import lax
from jax.experimental import pallas as pl
from jax.experimental.pallas import tpu as pltpu
from jax.experimental.pallas import tpu_sc as plsc

_EMB = 448


def _mean_body(nblk, inv_n, x_ref, o_ref, acc_ref):
    i = pl.program_id(0)

    @pl.when(i == 0)
    def _init():
        acc_ref[...] = jnp.zeros_like(acc_ref)

    acc_ref[...] += x_ref[...]

    @pl.when(i == nblk - 1)
    def _fin():
        o_ref[...] = acc_ref[...].sum(axis=(1, 2))[:, None] * inv_n


def _channel_means(image):
    c, h, w = image.shape
    bh = 16
    assert h % bh == 0
    nblk = h // bh
    return pl.pallas_call(
        functools.partial(_mean_body, nblk, 1.0 / (h * w)),
        grid=(nblk,),
        in_specs=[pl.BlockSpec((c, bh, w), lambda i: (0, i, 0))],
        out_specs=pl.BlockSpec((c, 1), lambda i: (0, 0)),
        out_shape=jax.ShapeDtypeStruct((c, 1), jnp.float32),
        scratch_shapes=[pltpu.VMEM((c, bh, w), jnp.float32)],
    )(image)


def _make_hist(n, nbins):
    info = plsc.get_sparse_core_info()
    nc, ns, nl = info.num_cores, info.num_subcores, info.num_lanes
    nw = nc * ns
    per_w = n // nw
    assert n % nw == 0 and per_w % nl == 0 and per_w % 8 == 0
    mesh = plsc.VectorSubcoreMesh(core_axis_name="c", subcore_axis_name="s")

    @functools.partial(
        pl.kernel,
        mesh=mesh,
        compiler_params=pltpu.CompilerParams(needs_layout_passes=False),
        out_type=jax.ShapeDtypeStruct((nw, nbins), jnp.float32),
        scratch_types=[
            pltpu.VMEM((per_w,), jnp.int32),
            pltpu.VMEM((nl * nbins,), jnp.float32),
            pltpu.VMEM((nbins,), jnp.float32),
        ],
    )
    def hist_kernel(lbl_hbm, out_hbm, lbl_v, hist_v, part_v):
        wid = lax.axis_index("s") * nc + lax.axis_index("c")
        base = wid * per_w
        pltpu.sync_copy(lbl_hbm.at[pl.ds(base, per_w)], lbl_v)

        def _zero(t, carry):
            hist_v[pl.ds(t * nl, nl)] = jnp.zeros((nl,), jnp.float32)
            return carry

        lax.fori_loop(0, (nl * nbins) // nl, _zero, 0)

        lane_base = lax.iota(jnp.int32, nl) * nbins
        ones = jnp.ones((nl,), jnp.float32)

        def _scat(j, carry):
            idx = lbl_v[pl.ds(j * nl, nl)]
            plsc.addupdate_scatter(hist_v, [lane_base + idx], ones)
            return carry

        lax.fori_loop(0, per_w // nl, _scat, 0)

        for cchunk in range(nbins // nl):
            acc = jnp.zeros((nl,), jnp.float32)
            for l in range(nl):
                acc = acc + hist_v[pl.ds(l * nbins + cchunk * nl, nl)]
            part_v[pl.ds(cchunk * nl, nl)] = acc

        pltpu.sync_copy(part_v, out_hbm.at[wid])

    return hist_kernel


def kernel(image, label):
    c, h, w = image.shape
    n = h * w
    nbins = _EMB - c
    mean_c = _channel_means(image)
    parts = _make_hist(n, nbins)(label.reshape(n))
    hist = parts.sum(axis=0) * (1.0 / n)
    return jnp.concatenate([mean_c[:, 0], hist])


# speedup vs baseline: 17.2373x; 1.0602x over previous
import functools

import jax
import jax.numpy as jnp
from jax import lax
from jax.experimental import pallas as pl
from jax.experimental.pallas import tpu as pltpu
from jax.experimental.pallas import tpu_sc as plsc

_EMB = 448


def _mean_body(nblk, inv_n, x_ref, o_ref, acc_ref):
    i = pl.program_id(0)

    @pl.when(i == 0)
    def _init():
        acc_ref[...] = jnp.zeros_like(acc_ref)

    acc_ref[...] += x_ref[...]

    @pl.when(i == nblk - 1)
    def _fin():
        o_ref[...] = acc_ref[...].sum(axis=(1, 2))[:, None] * inv_n


def _channel_means(image):
    c, h, w = image.shape
    bh = 32
    assert h % bh == 0
    nblk = h // bh
    return pl.pallas_call(
        functools.partial(_mean_body, nblk, 1.0 / (h * w)),
        grid=(nblk,),
        in_specs=[pl.BlockSpec((c, bh, w), lambda i: (0, i, 0))],
        out_specs=pl.BlockSpec((c, 1), lambda i: (0, 0)),
        out_shape=jax.ShapeDtypeStruct((c, 1), jnp.float32),
        scratch_shapes=[pltpu.VMEM((c, bh, w), jnp.float32)],
    )(image)


def _make_hist(n, nbins):
    info = plsc.get_sparse_core_info()
    nc, ns, nl = info.num_cores, info.num_subcores, info.num_lanes
    nw = nc * ns
    per_w = n // nw
    assert n % nw == 0 and per_w % nl == 0 and per_w % 8 == 0
    mesh = plsc.VectorSubcoreMesh(core_axis_name="c", subcore_axis_name="s")

    @functools.partial(
        pl.kernel,
        mesh=mesh,
        compiler_params=pltpu.CompilerParams(needs_layout_passes=False),
        out_type=jax.ShapeDtypeStruct((nw, nbins), jnp.float32),
        scratch_types=[
            pltpu.VMEM((per_w,), jnp.int32),
            pltpu.VMEM((nl * nbins,), jnp.float32),
            pltpu.VMEM((nbins,), jnp.float32),
        ],
    )
    def hist_kernel(lbl_hbm, out_hbm, lbl_v, hist_v, part_v):
        wid = lax.axis_index("s") * nc + lax.axis_index("c")
        base = wid * per_w
        pltpu.sync_copy(lbl_hbm.at[pl.ds(base, per_w)], lbl_v)

        def _zero(t, carry):
            hist_v[pl.ds(t * nl, nl)] = jnp.zeros((nl,), jnp.float32)
            return carry

        lax.fori_loop(0, (nl * nbins) // nl, _zero, 0)

        lane_base = lax.iota(jnp.int32, nl) * nbins
        ones = jnp.ones((nl,), jnp.float32)

        def _scat(j, carry):
            idx = lbl_v[pl.ds(j * nl, nl)]
            plsc.addupdate_scatter(hist_v, [lane_base + idx], ones)
            return carry

        lax.fori_loop(0, per_w // nl, _scat, 0)

        for cchunk in range(nbins // nl):
            acc = jnp.zeros((nl,), jnp.float32)
            for l in range(nl):
                acc = acc + hist_v[pl.ds(l * nbins + cchunk * nl, nl)]
            part_v[pl.ds(cchunk * nl, nl)] = acc

        pltpu.sync_copy(part_v, out_hbm.at[wid])

    return hist_kernel


def kernel(image, label):
    c, h, w = image.shape
    n = h * w
    nbins = _EMB - c
    mean_c = _channel_means(image)
    parts = _make_hist(n, nbins)(label.reshape(n))
    hist = parts.sum(axis=0) * (1.0 / n)
    return jnp.concatenate([mean_c[:, 0], hist])
